# baseline (speedup 1.0000x reference)
import functools

import jax
import jax.numpy as jnp
from jax import lax
from jax.experimental import pallas as pl
from jax.experimental.pallas import tpu as pltpu
from jax.experimental.pallas import tpu_sc as plsc

N = 10000
E = 320000
D_IN = 128
D_HID = 16
D_OUT = 128

NP_ = 10240
NC, NS = 2, 16
NW = NC * NS
CB = 128
E_PAD = 327680
NCHUNK = E_PAD // (NW * CB)
ROWS_PER_TILE = NP_ // NS


def _mesh():
    return plsc.VectorSubcoreMesh(core_axis_name="c", subcore_axis_name="s")


@functools.partial(
    pl.kernel,
    out_type=(jax.ShapeDtypeStruct((NC, NP_, D_HID), jnp.float32),
              jax.ShapeDtypeStruct((NP_, D_HID), jnp.float32),
              jax.ShapeDtypeStruct((NP_, D_HID), jnp.float32)),
    mesh=_mesh(),
    compiler_params=pltpu.CompilerParams(use_tc_tiling_on_sc=False,
                                         needs_layout_passes=False),
    scratch_types=[
        pltpu.VMEM((NCHUNK, CB), jnp.int32),
        pltpu.VMEM((NCHUNK, CB), jnp.int32),
        pltpu.VMEM((NCHUNK, CB), jnp.int32),
        pltpu.VMEM((CB,), jnp.float32),
        pltpu.VMEM((CB,), jnp.float32),
        [pltpu.VMEM((CB, D_HID), jnp.float32)] * 8,
        pltpu.VMEM((CB, D_HID), jnp.float32),
        pltpu.VMEM((ROWS_PER_TILE,), jnp.float32),
        pltpu.VMEM((ROWS_PER_TILE, D_HID), jnp.float32),
        pltpu.VMEM((ROWS_PER_TILE, D_HID), jnp.float32),
        pltpu.VMEM((ROWS_PER_TILE, D_HID), jnp.float32),
        pltpu.VMEM_SHARED((NP_,), jnp.float32),
        pltpu.VMEM_SHARED((NP_, D_HID), jnp.float32),
        pltpu.VMEM_SHARED((NP_, D_HID), jnp.float32),
        [pltpu.SemaphoreType.DMA] * 8,
        [pltpu.SemaphoreType.DMA] * 8,
        pltpu.SemaphoreType.DMA,
    ],
)
def _pre_agg_kernel(xw_hbm, src_hbm, dst_hbm, out_hbm, xws_hbm, dis16_hbm,
                    src_v, dst_v, dstb_v, ones_v, zeros1_v, rows, zero_v,
                    deg_v, xw_v, xws_v, dis16_v, deg_sh, acc_sh, tab_sh,
                    gsem, ssem, dsem):
    NSLOT, LA = 8, 4
    c = lax.axis_index("c")
    s = lax.axis_index("s")
    wid = s * NC + c
    row0 = s * ROWS_PER_TILE

    for i in range(CB):
        zero_v[i, :] = jnp.zeros((D_HID,), jnp.float32)
    for i in range(CB // 16):
        ones_v[pl.ds(i * 16, 16)] = jnp.ones((16,), jnp.float32)
        zeros1_v[pl.ds(i * 16, 16)] = jnp.zeros((16,), jnp.float32)
    for i in range(ROWS_PER_TILE // CB):
        pltpu.sync_copy(zero_v, acc_sh.at[pl.ds(row0 + i * CB, CB)])
        pltpu.sync_copy(zeros1_v, deg_sh.at[pl.ds(row0 + i * CB, CB)])
    plsc.subcore_barrier()

    pltpu.sync_copy(src_hbm.at[wid], src_v)
    pltpu.sync_copy(dst_hbm.at[wid], dst_v)
    pltpu.sync_copy(dst_hbm.at[s * NC + (1 - c)], dstb_v)

    def dbody(j, carry):
        pltpu.async_copy(ones_v, deg_sh.at[dst_v.at[j]], dsem, add=True)
        pltpu.async_copy(ones_v, deg_sh.at[dstb_v.at[j]], dsem, add=True)
        return carry

    lax.fori_loop(0, NCHUNK, dbody, 0)
    pltpu.make_async_copy(dst_hbm.at[wid], dstb_v, dsem).wait()
    pltpu.make_async_copy(dst_hbm.at[wid], dstb_v, dsem).wait()
    plsc.subcore_barrier()

    pltpu.sync_copy(deg_sh.at[pl.ds(row0, ROWS_PER_TILE)], deg_v)
    pltpu.sync_copy(xw_hbm.at[pl.ds(row0, ROWS_PER_TILE)], xw_v)
    for i in range(ROWS_PER_TILE // 16):
        d = deg_v[pl.ds(i * 16, 16)] + 1.0
        yi = 0x5F3759DF - lax.shift_right_logical(
            plsc.bitcast(d, jnp.int32), 1)
        y = plsc.bitcast(yi, jnp.float32)
        hd = d * 0.5
        y = y * (1.5 - hd * y * y)
        y = y * (1.5 - hd * y * y)
        y = y * (1.5 - hd * y * y)
        deg_v[pl.ds(i * 16, 16)] = y

    def sbody(r4, carry):
        for u in range(4):
            r = r4 * 4 + u
            dsc = plsc.load_gather(deg_v, [jnp.full((16,), r, jnp.int32)])
            dis16_v[r, :] = dsc
            xws_v[r, :] = xw_v[r, :] * dsc
        return carry

    lax.fori_loop(0, ROWS_PER_TILE // 4, sbody, 0)
    pltpu.sync_copy(xws_v, tab_sh.at[pl.ds(row0, ROWS_PER_TILE)])

    @pl.when(c == 0)
    def _():
        pltpu.sync_copy(xws_v, xws_hbm.at[pl.ds(row0, ROWS_PER_TILE)])
        pltpu.sync_copy(dis16_v, dis16_hbm.at[pl.ds(row0, ROWS_PER_TILE)])

    plsc.subcore_barrier()

    for j in range(LA):
        pltpu.async_copy(tab_sh.at[src_v.at[j]], rows[j], gsem[j])

    def body(g, carry):
        for b in range(NSLOT):
            j = g * NSLOT + b
            b2 = (b + LA) % NSLOT
            pltpu.make_async_copy(
                xw_hbm.at[pl.ds(0, CB)], rows[b], gsem[b]).wait()
            pltpu.async_copy(rows[b], acc_sh.at[dst_v.at[j]], ssem[b],
                             add=True)

            @pl.when((j >= NSLOT - LA) & (j < NCHUNK - LA))
            def _():
                pltpu.make_async_copy(
                    xw_hbm.at[pl.ds(0, CB)], rows[b2], ssem[b2]).wait()

            @pl.when(j < NCHUNK - LA)
            def _():
                pltpu.async_copy(
                    tab_sh.at[src_v.at[j + LA]], rows[b2], gsem[b2])
        return carry

    lax.fori_loop(0, NCHUNK // NSLOT, body, 0)
    for b in range(NSLOT):
        pltpu.make_async_copy(
            xw_hbm.at[pl.ds(0, CB)], rows[b], ssem[b]).wait()
    plsc.subcore_barrier()
    pltpu.sync_copy(acc_sh.at[pl.ds(row0, ROWS_PER_TILE)],
                    out_hbm.at[c, pl.ds(row0, ROWS_PER_TILE)])


@functools.partial(
    pl.kernel,
    out_type=(jax.ShapeDtypeStruct((NC, NP_, D_HID), jnp.float32),
              jax.ShapeDtypeStruct((NP_, D_HID), jnp.float32)),
    mesh=_mesh(),
    compiler_params=pltpu.CompilerParams(use_tc_tiling_on_sc=False,
                                         needs_layout_passes=False),
    scratch_types=[
        pltpu.VMEM((NCHUNK, CB), jnp.int32),
        pltpu.VMEM((NCHUNK, CB), jnp.int32),
        [pltpu.VMEM((CB, D_HID), jnp.float32)] * 8,
        pltpu.VMEM((CB, D_HID), jnp.float32),
        [pltpu.VMEM((ROWS_PER_TILE, D_HID), jnp.float32)] * 4,
        pltpu.VMEM((ROWS_PER_TILE, D_HID), jnp.float32),
        pltpu.VMEM((D_HID,), jnp.float32),
        pltpu.VMEM_SHARED((NP_, D_HID), jnp.float32),
        pltpu.VMEM_SHARED((NP_, D_HID), jnp.float32),
        [pltpu.SemaphoreType.DMA] * 8,
        [pltpu.SemaphoreType.DMA] * 8,
    ],
)
def _relu_agg_kernel(p_hbm, xws_hbm, dis16_hbm, b1_hbm, src_hbm, dst_hbm,
                     out_hbm, hs_hbm, src_v, dst_v, rows, zero_v, bufs,
                     hrow_v, b1_v, acc_sh, htab_sh, gsem, ssem):
    NSLOT, LA = 8, 4
    c = lax.axis_index("c")
    s = lax.axis_index("s")
    wid = s * NC + c
    row0 = s * ROWS_PER_TILE
    bp0, bp1, bxw, bdis = bufs

    for i in range(CB):
        zero_v[i, :] = jnp.zeros((D_HID,), jnp.float32)
    for i in range(ROWS_PER_TILE // CB):
        pltpu.sync_copy(zero_v, acc_sh.at[pl.ds(row0 + i * CB, CB)])

    pltpu.sync_copy(b1_hbm, b1_v)
    pltpu.sync_copy(p_hbm.at[0, pl.ds(row0, ROWS_PER_TILE)], bp0)
    pltpu.sync_copy(p_hbm.at[1, pl.ds(row0, ROWS_PER_TILE)], bp1)
    pltpu.sync_copy(xws_hbm.at[pl.ds(row0, ROWS_PER_TILE)], bxw)
    pltpu.sync_copy(dis16_hbm.at[pl.ds(row0, ROWS_PER_TILE)], bdis)
    b1r = b1_v[...]

    def rbody(r4, carry):
        for u in range(4):
            r = r4 * 4 + u
            d = bdis[r, :]
            t = (bp0[r, :] + bp1[r, :] + bxw[r, :]) * d + b1r
            hrow_v[r, :] = jnp.maximum(t, 0.0) * d
        return carry

    lax.fori_loop(0, ROWS_PER_TILE // 4, rbody, 0)
    pltpu.sync_copy(hrow_v, htab_sh.at[pl.ds(row0, ROWS_PER_TILE)])

    @pl.when(c == 0)
    def _():
        pltpu.sync_copy(hrow_v, hs_hbm.at[pl.ds(row0, ROWS_PER_TILE)])

    plsc.subcore_barrier()

    pltpu.sync_copy(src_hbm.at[wid], src_v)
    pltpu.sync_copy(dst_hbm.at[wid], dst_v)

    for j in range(LA):
        pltpu.async_copy(htab_sh.at[src_v.at[j]], rows[j], gsem[j])

    def body(g, carry):
        for b in range(NSLOT):
            j = g * NSLOT + b
            b2 = (b + LA) % NSLOT
            pltpu.make_async_copy(
                p_hbm.at[0, pl.ds(0, CB)], rows[b], gsem[b]).wait()
            pltpu.async_copy(rows[b], acc_sh.at[dst_v.at[j]], ssem[b],
                             add=True)

            @pl.when((j >= NSLOT - LA) & (j < NCHUNK - LA))
            def _():
                pltpu.make_async_copy(
                    p_hbm.at[0, pl.ds(0, CB)], rows[b2], ssem[b2]).wait()

            @pl.when(j < NCHUNK - LA)
            def _():
                pltpu.async_copy(
                    htab_sh.at[src_v.at[j + LA]], rows[b2], gsem[b2])
        return carry

    lax.fori_loop(0, NCHUNK // NSLOT, body, 0)
    for b in range(NSLOT):
        pltpu.make_async_copy(
            p_hbm.at[0, pl.ds(0, CB)], rows[b], ssem[b]).wait()
    plsc.subcore_barrier()
    pltpu.sync_copy(acc_sh.at[pl.ds(row0, ROWS_PER_TILE)],
                    out_hbm.at[c, pl.ds(row0, ROWS_PER_TILE)])


def _mm1_body(x_ref, w1_ref, xw_ref):
    xw = jnp.dot(x_ref[...], w1_ref[...], preferred_element_type=jnp.float32)
    xw_ref[pl.ds(0, N), :] = xw
    xw_ref[pl.ds(N, NP_ - N), :] = jnp.zeros((NP_ - N, D_HID), jnp.float32)


def _k5_body(q_ref, hs_ref, dis16_ref, w2_ref, b2_ref, out_ref):
    z = (q_ref[0] + q_ref[1] + hs_ref[...]) * dis16_ref[...]
    out_ref[...] = (
        jnp.dot(z[:N], w2_ref[...], preferred_element_type=jnp.float32)
        + b2_ref[...][None, :]
    )


def kernel(x, edge_index, W1, b1, W2, b2):
    pad_idx = (jnp.arange(E_PAD - E, dtype=jnp.int32) % (NP_ - N)) + N
    src = jnp.concatenate([edge_index[0], pad_idx]).reshape(NW, NCHUNK, CB)
    dst = jnp.concatenate([edge_index[1], pad_idx]).reshape(NW, NCHUNK, CB)

    xw = pl.pallas_call(
        _mm1_body,
        out_shape=jax.ShapeDtypeStruct((NP_, D_HID), jnp.float32),
        in_specs=[pl.BlockSpec(memory_space=pltpu.VMEM)] * 2,
        out_specs=pl.BlockSpec(memory_space=pltpu.VMEM),
    )(x, W1)

    p, xw_s, dis16 = _pre_agg_kernel(xw, src, dst)

    q, h_s = _relu_agg_kernel(p, xw_s, dis16, b1, src, dst)

    out = pl.pallas_call(
        _k5_body,
        out_shape=jax.ShapeDtypeStruct((N, D_OUT), jnp.float32),
        in_specs=[pl.BlockSpec(memory_space=pltpu.VMEM)] * 5,
        out_specs=pl.BlockSpec(memory_space=pltpu.VMEM),
    )(q, h_s, dis16, W2, b2)

    return out


# speedup vs baseline: 1.0190x; 1.0190x over previous
import functools

import jax
import jax.numpy as jnp
from jax import lax
from jax.experimental import pallas as pl
from jax.experimental.pallas import tpu as pltpu
from jax.experimental.pallas import tpu_sc as plsc

N = 10000
E = 320000
D_IN = 128
D_HID = 16
D_OUT = 128

NP_ = 10240
NC, NS = 2, 16
NW = NC * NS
CB = 128
E_PAD = 327680
NCHUNK = E_PAD // (NW * CB)
ROWS_PER_TILE = NP_ // NS


def _mesh():
    return plsc.VectorSubcoreMesh(core_axis_name="c", subcore_axis_name="s")


@functools.partial(
    pl.kernel,
    out_type=(jax.ShapeDtypeStruct((NC, NP_, D_HID), jnp.float32),
              jax.ShapeDtypeStruct((NP_, D_HID), jnp.float32),
              jax.ShapeDtypeStruct((NP_, D_HID), jnp.float32)),
    mesh=_mesh(),
    compiler_params=pltpu.CompilerParams(use_tc_tiling_on_sc=False,
                                         needs_layout_passes=False),
    scratch_types=[
        pltpu.VMEM((NCHUNK, 2, CB), jnp.int32),
        pltpu.VMEM((NCHUNK, 2, CB), jnp.int32),
        pltpu.VMEM((CB,), jnp.float32),
        pltpu.VMEM((CB,), jnp.float32),
        [pltpu.VMEM((CB, D_HID), jnp.float32)] * 8,
        pltpu.VMEM((CB, D_HID), jnp.float32),
        pltpu.VMEM((ROWS_PER_TILE,), jnp.float32),
        pltpu.VMEM((ROWS_PER_TILE, D_HID), jnp.float32),
        pltpu.VMEM((ROWS_PER_TILE, D_HID), jnp.float32),
        pltpu.VMEM((ROWS_PER_TILE, D_HID), jnp.float32),
        pltpu.VMEM_SHARED((NP_,), jnp.float32),
        pltpu.VMEM_SHARED((NP_, D_HID), jnp.float32),
        pltpu.VMEM_SHARED((NP_, D_HID), jnp.float32),
        [pltpu.SemaphoreType.DMA] * 8,
        [pltpu.SemaphoreType.DMA] * 8,
        pltpu.SemaphoreType.DMA,
    ],
)
def _pre_agg_kernel(xw_hbm, edges_hbm, out_hbm, xws_hbm, dis16_hbm,
                    idx_v, idxb_v, ones_v, zeros1_v, rows, zero_v,
                    deg_v, xw_v, xws_v, dis16_v, deg_sh, acc_sh, tab_sh,
                    gsem, ssem, dsem):
    NSLOT, LA = 8, 4
    c = lax.axis_index("c")
    s = lax.axis_index("s")
    wid = s * NC + c
    row0 = s * ROWS_PER_TILE

    for i in range(CB):
        zero_v[i, :] = jnp.zeros((D_HID,), jnp.float32)
    for i in range(CB // 16):
        ones_v[pl.ds(i * 16, 16)] = jnp.ones((16,), jnp.float32)
        zeros1_v[pl.ds(i * 16, 16)] = jnp.zeros((16,), jnp.float32)
    for i in range(ROWS_PER_TILE // CB):
        pltpu.sync_copy(zero_v, acc_sh.at[pl.ds(row0 + i * CB, CB)])
        pltpu.sync_copy(zeros1_v, deg_sh.at[pl.ds(row0 + i * CB, CB)])
    plsc.subcore_barrier()

    pltpu.sync_copy(edges_hbm.at[wid], idx_v)
    pltpu.sync_copy(edges_hbm.at[s * NC + (1 - c)], idxb_v)

    def dbody(j, carry):
        pltpu.async_copy(ones_v, deg_sh.at[idx_v.at[j, 1]], dsem, add=True)
        pltpu.async_copy(ones_v, deg_sh.at[idxb_v.at[j, 1]], dsem, add=True)
        return carry

    lax.fori_loop(0, NCHUNK, dbody, 0)
    pltpu.make_async_copy(edges_hbm.at[wid], idxb_v, dsem).wait()
    plsc.subcore_barrier()

    pltpu.sync_copy(deg_sh.at[pl.ds(row0, ROWS_PER_TILE)], deg_v)
    pltpu.sync_copy(xw_hbm.at[pl.ds(row0, ROWS_PER_TILE)], xw_v)
    for i in range(ROWS_PER_TILE // 16):
        d = deg_v[pl.ds(i * 16, 16)] + 1.0
        yi = 0x5F3759DF - lax.shift_right_logical(
            plsc.bitcast(d, jnp.int32), 1)
        y = plsc.bitcast(yi, jnp.float32)
        hd = d * 0.5
        y = y * (1.5 - hd * y * y)
        y = y * (1.5 - hd * y * y)
        y = y * (1.5 - hd * y * y)
        deg_v[pl.ds(i * 16, 16)] = y

    def sbody(r4, carry):
        for u in range(4):
            r = r4 * 4 + u
            dsc = plsc.load_gather(deg_v, [jnp.full((16,), r, jnp.int32)])
            dis16_v[r, :] = dsc
            xws_v[r, :] = xw_v[r, :] * dsc
        return carry

    lax.fori_loop(0, ROWS_PER_TILE // 4, sbody, 0)
    pltpu.sync_copy(xws_v, tab_sh.at[pl.ds(row0, ROWS_PER_TILE)])

    @pl.when(c == 0)
    def _():
        pltpu.sync_copy(xws_v, xws_hbm.at[pl.ds(row0, ROWS_PER_TILE)])
        pltpu.sync_copy(dis16_v, dis16_hbm.at[pl.ds(row0, ROWS_PER_TILE)])

    plsc.subcore_barrier()

    for j in range(LA):
        pltpu.async_copy(tab_sh.at[idx_v.at[j, 0]], rows[j], gsem[j])

    def body(g, carry):
        for b in range(NSLOT):
            j = g * NSLOT + b
            b2 = (b + LA) % NSLOT
            pltpu.make_async_copy(
                xw_hbm.at[pl.ds(0, CB)], rows[b], gsem[b]).wait()
            pltpu.async_copy(rows[b], acc_sh.at[idx_v.at[j, 1]], ssem[b],
                             add=True)

            @pl.when((j >= NSLOT - LA) & (j < NCHUNK - LA))
            def _():
                pltpu.make_async_copy(
                    xw_hbm.at[pl.ds(0, CB)], rows[b2], ssem[b2]).wait()

            @pl.when(j < NCHUNK - LA)
            def _():
                pltpu.async_copy(
                    tab_sh.at[idx_v.at[j + LA, 0]], rows[b2], gsem[b2])
        return carry

    lax.fori_loop(0, NCHUNK // NSLOT, body, 0)
    for b in range(NSLOT):
        pltpu.make_async_copy(
            xw_hbm.at[pl.ds(0, CB)], rows[b], ssem[b]).wait()
    plsc.subcore_barrier()
    pltpu.sync_copy(acc_sh.at[pl.ds(row0, ROWS_PER_TILE)],
                    out_hbm.at[c, pl.ds(row0, ROWS_PER_TILE)])


@functools.partial(
    pl.kernel,
    out_type=(jax.ShapeDtypeStruct((NC, NP_, D_HID), jnp.float32),
              jax.ShapeDtypeStruct((NP_, D_HID), jnp.float32)),
    mesh=_mesh(),
    compiler_params=pltpu.CompilerParams(use_tc_tiling_on_sc=False,
                                         needs_layout_passes=False),
    scratch_types=[
        pltpu.VMEM((NCHUNK, 2, CB), jnp.int32),
        [pltpu.VMEM((CB, D_HID), jnp.float32)] * 8,
        pltpu.VMEM((CB, D_HID), jnp.float32),
        [pltpu.VMEM((ROWS_PER_TILE, D_HID), jnp.float32)] * 4,
        pltpu.VMEM((ROWS_PER_TILE, D_HID), jnp.float32),
        pltpu.VMEM((D_HID,), jnp.float32),
        pltpu.VMEM_SHARED((NP_, D_HID), jnp.float32),
        pltpu.VMEM_SHARED((NP_, D_HID), jnp.float32),
        [pltpu.SemaphoreType.DMA] * 8,
        [pltpu.SemaphoreType.DMA] * 8,
    ],
)
def _relu_agg_kernel(p_hbm, xws_hbm, dis16_hbm, b1_hbm, edges_hbm,
                     out_hbm, hs_hbm, idx_v, rows, zero_v, bufs,
                     hrow_v, b1_v, acc_sh, htab_sh, gsem, ssem):
    NSLOT, LA = 8, 4
    c = lax.axis_index("c")
    s = lax.axis_index("s")
    wid = s * NC + c
    row0 = s * ROWS_PER_TILE
    bp0, bp1, bxw, bdis = bufs

    for i in range(CB):
        zero_v[i, :] = jnp.zeros((D_HID,), jnp.float32)
    for i in range(ROWS_PER_TILE // CB):
        pltpu.sync_copy(zero_v, acc_sh.at[pl.ds(row0 + i * CB, CB)])

    pltpu.sync_copy(b1_hbm, b1_v)
    pltpu.sync_copy(p_hbm.at[0, pl.ds(row0, ROWS_PER_TILE)], bp0)
    pltpu.sync_copy(p_hbm.at[1, pl.ds(row0, ROWS_PER_TILE)], bp1)
    pltpu.sync_copy(xws_hbm.at[pl.ds(row0, ROWS_PER_TILE)], bxw)
    pltpu.sync_copy(dis16_hbm.at[pl.ds(row0, ROWS_PER_TILE)], bdis)
    b1r = b1_v[...]

    def rbody(r4, carry):
        for u in range(4):
            r = r4 * 4 + u
            d = bdis[r, :]
            t = (bp0[r, :] + bp1[r, :] + bxw[r, :]) * d + b1r
            hrow_v[r, :] = jnp.maximum(t, 0.0) * d
        return carry

    lax.fori_loop(0, ROWS_PER_TILE // 4, rbody, 0)
    pltpu.sync_copy(hrow_v, htab_sh.at[pl.ds(row0, ROWS_PER_TILE)])

    @pl.when(c == 0)
    def _():
        pltpu.sync_copy(hrow_v, hs_hbm.at[pl.ds(row0, ROWS_PER_TILE)])

    plsc.subcore_barrier()

    pltpu.sync_copy(edges_hbm.at[wid], idx_v)

    for j in range(LA):
        pltpu.async_copy(htab_sh.at[idx_v.at[j, 0]], rows[j], gsem[j])

    def body(g, carry):
        for b in range(NSLOT):
            j = g * NSLOT + b
            b2 = (b + LA) % NSLOT
            pltpu.make_async_copy(
                p_hbm.at[0, pl.ds(0, CB)], rows[b], gsem[b]).wait()
            pltpu.async_copy(rows[b], acc_sh.at[idx_v.at[j, 1]], ssem[b],
                             add=True)

            @pl.when((j >= NSLOT - LA) & (j < NCHUNK - LA))
            def _():
                pltpu.make_async_copy(
                    p_hbm.at[0, pl.ds(0, CB)], rows[b2], ssem[b2]).wait()

            @pl.when(j < NCHUNK - LA)
            def _():
                pltpu.async_copy(
                    htab_sh.at[idx_v.at[j + LA, 0]], rows[b2], gsem[b2])
        return carry

    lax.fori_loop(0, NCHUNK // NSLOT, body, 0)
    for b in range(NSLOT):
        pltpu.make_async_copy(
            p_hbm.at[0, pl.ds(0, CB)], rows[b], ssem[b]).wait()
    plsc.subcore_barrier()
    pltpu.sync_copy(acc_sh.at[pl.ds(row0, ROWS_PER_TILE)],
                    out_hbm.at[c, pl.ds(row0, ROWS_PER_TILE)])


def _mm1_body(x_ref, w1_ref, xw_ref):
    xw = jnp.dot(x_ref[...], w1_ref[...], preferred_element_type=jnp.float32)
    xw_ref[pl.ds(0, N), :] = xw
    xw_ref[pl.ds(N, NP_ - N), :] = jnp.zeros((NP_ - N, D_HID), jnp.float32)


def _k5_body(q_ref, hs_ref, dis16_ref, w2_ref, b2_ref, out_ref):
    z = (q_ref[0] + q_ref[1] + hs_ref[...]) * dis16_ref[...]
    out_ref[...] = (
        jnp.dot(z[:N], w2_ref[...], preferred_element_type=jnp.float32)
        + b2_ref[...][None, :]
    )


def kernel(x, edge_index, W1, b1, W2, b2):
    ei = jnp.transpose(edge_index.reshape(2, E // CB, CB), (1, 0, 2))
    padblk = ((jnp.arange((E_PAD - E) // CB * 2 * CB, dtype=jnp.int32)
               % (NP_ - N)) + N).reshape((E_PAD - E) // CB, 2, CB)
    edges = jnp.concatenate([ei, padblk]).reshape(NW, NCHUNK, 2, CB)

    xw = pl.pallas_call(
        _mm1_body,
        out_shape=jax.ShapeDtypeStruct((NP_, D_HID), jnp.float32),
        in_specs=[pl.BlockSpec(memory_space=pltpu.VMEM)] * 2,
        out_specs=pl.BlockSpec(memory_space=pltpu.VMEM),
    )(x, W1)

    p, xw_s, dis16 = _pre_agg_kernel(xw, edges)

    q, h_s = _relu_agg_kernel(p, xw_s, dis16, b1, edges)

    out = pl.pallas_call(
        _k5_body,
        out_shape=jax.ShapeDtypeStruct((N, D_OUT), jnp.float32),
        in_specs=[pl.BlockSpec(memory_space=pltpu.VMEM)] * 5,
        out_specs=pl.BlockSpec(memory_space=pltpu.VMEM),
    )(q, h_s, dis16, W2, b2)

    return out


# speedup vs baseline: 1.0872x; 1.0670x over previous
import functools

import jax
import jax.numpy as jnp
from jax import lax
from jax.experimental import pallas as pl
from jax.experimental.pallas import tpu as pltpu
from jax.experimental.pallas import tpu_sc as plsc

N = 10000
E = 320000
D_IN = 128
D_HID = 16
D_OUT = 128

NP_ = 10240
NC, NS = 2, 16
NW = NC * NS
CB = 128
E_PAD = 327680
NCHUNK = E_PAD // (NW * CB)
ROWS_PER_TILE = NP_ // NS


def _mesh():
    return plsc.VectorSubcoreMesh(core_axis_name="c", subcore_axis_name="s")


@functools.partial(
    pl.kernel,
    out_type=(jax.ShapeDtypeStruct((NC, NP_, D_HID), jnp.float32),
              jax.ShapeDtypeStruct((NP_, D_HID), jnp.float32),
              jax.ShapeDtypeStruct((NP_, D_HID), jnp.float32)),
    mesh=_mesh(),
    compiler_params=pltpu.CompilerParams(use_tc_tiling_on_sc=False,
                                         needs_layout_passes=False),
    scratch_types=[
        pltpu.VMEM((NCHUNK, 2, CB), jnp.int32),
        pltpu.VMEM((NCHUNK, 2, CB), jnp.int32),
        pltpu.VMEM((CB,), jnp.float32),
        pltpu.VMEM((CB,), jnp.float32),
        [pltpu.VMEM((CB, D_HID), jnp.float32)] * 8,
        pltpu.VMEM((CB, D_HID), jnp.float32),
        pltpu.VMEM((ROWS_PER_TILE,), jnp.float32),
        pltpu.VMEM((ROWS_PER_TILE, D_HID), jnp.float32),
        pltpu.VMEM((ROWS_PER_TILE, D_HID), jnp.float32),
        pltpu.VMEM((ROWS_PER_TILE, D_HID), jnp.float32),
        pltpu.VMEM_SHARED((NP_,), jnp.float32),
        pltpu.VMEM_SHARED((NP_, D_HID), jnp.float32),
        pltpu.VMEM_SHARED((NP_, D_HID), jnp.float32),
        [pltpu.SemaphoreType.DMA] * 8,
        [pltpu.SemaphoreType.DMA] * 8,
        pltpu.SemaphoreType.DMA,
    ],
)
def _pre_agg_kernel(xw_hbm, edges_hbm, out_hbm, xws_hbm, dis16_hbm,
                    idx_v, idxb_v, ones_v, zeros1_v, rows, zero_v,
                    deg_v, xw_v, xws_v, dis16_v, deg_sh, acc_sh, tab_sh,
                    gsem, ssem, dsem):
    NSLOT, LA = 8, 4
    c = lax.axis_index("c")
    s = lax.axis_index("s")
    wid = s * NC + c
    row0 = s * ROWS_PER_TILE

    for i in range(CB):
        zero_v[i, :] = jnp.zeros((D_HID,), jnp.float32)
    for i in range(CB // 16):
        ones_v[pl.ds(i * 16, 16)] = jnp.ones((16,), jnp.float32)
        zeros1_v[pl.ds(i * 16, 16)] = jnp.zeros((16,), jnp.float32)
    for i in range(ROWS_PER_TILE // CB):
        pltpu.sync_copy(zero_v, acc_sh.at[pl.ds(row0 + i * CB, CB)])
        pltpu.sync_copy(zeros1_v, deg_sh.at[pl.ds(row0 + i * CB, CB)])
    plsc.subcore_barrier()

    pltpu.sync_copy(edges_hbm.at[wid], idx_v)
    pltpu.sync_copy(edges_hbm.at[s * NC + (1 - c)], idxb_v)

    def dbody(j, carry):
        pltpu.async_copy(ones_v, deg_sh.at[idx_v.at[j, 1]], dsem, add=True)
        pltpu.async_copy(ones_v, deg_sh.at[idxb_v.at[j, 1]], dsem, add=True)
        return carry

    lax.fori_loop(0, NCHUNK, dbody, 0)
    pltpu.make_async_copy(edges_hbm.at[wid], idxb_v, dsem).wait()
    plsc.subcore_barrier()

    pltpu.sync_copy(deg_sh.at[pl.ds(row0, ROWS_PER_TILE)], deg_v)
    pltpu.sync_copy(xw_hbm.at[pl.ds(row0, ROWS_PER_TILE)], xw_v)
    for i in range(ROWS_PER_TILE // 16):
        d = deg_v[pl.ds(i * 16, 16)] + 1.0
        yi = 0x5F3759DF - lax.shift_right_logical(
            plsc.bitcast(d, jnp.int32), 1)
        y = plsc.bitcast(yi, jnp.float32)
        hd = d * 0.5
        y = y * (1.5 - hd * y * y)
        y = y * (1.5 - hd * y * y)
        y = y * (1.5 - hd * y * y)
        deg_v[pl.ds(i * 16, 16)] = y

    def sbody(r4, carry):
        for u in range(4):
            r = r4 * 4 + u
            dsc = plsc.load_gather(deg_v, [jnp.full((16,), r, jnp.int32)])
            dis16_v[r, :] = dsc
            xws_v[r, :] = xw_v[r, :] * dsc
        return carry

    lax.fori_loop(0, ROWS_PER_TILE // 4, sbody, 0)
    pltpu.sync_copy(xws_v, tab_sh.at[pl.ds(row0, ROWS_PER_TILE)])

    @pl.when(c == 0)
    def _():
        pltpu.sync_copy(xws_v, xws_hbm.at[pl.ds(row0, ROWS_PER_TILE)])
        pltpu.sync_copy(dis16_v, dis16_hbm.at[pl.ds(row0, ROWS_PER_TILE)])

    plsc.subcore_barrier()

    for j in range(LA):
        pltpu.async_copy(tab_sh.at[idx_v.at[j, 0]], rows[j], gsem[j])

    def body(g, carry):
        for b in range(NSLOT):
            j = g * NSLOT + b
            b2 = (b + LA) % NSLOT
            pltpu.make_async_copy(
                xw_hbm.at[pl.ds(0, CB)], rows[b], gsem[b]).wait()
            pltpu.async_copy(rows[b], acc_sh.at[idx_v.at[j, 1]], ssem[b],
                             add=True)

            @pl.when((j >= NSLOT - LA) & (j < NCHUNK - LA))
            def _():
                pltpu.make_async_copy(
                    xw_hbm.at[pl.ds(0, CB)], rows[b2], ssem[b2]).wait()

            @pl.when(j < NCHUNK - LA)
            def _():
                pltpu.async_copy(
                    tab_sh.at[idx_v.at[j + LA, 0]], rows[b2], gsem[b2])
        return carry

    lax.fori_loop(0, NCHUNK // NSLOT, body, 0)
    for b in range(NSLOT):
        pltpu.make_async_copy(
            xw_hbm.at[pl.ds(0, CB)], rows[b], ssem[b]).wait()
    plsc.subcore_barrier()
    pltpu.sync_copy(acc_sh.at[pl.ds(row0, ROWS_PER_TILE)],
                    out_hbm.at[c, pl.ds(row0, ROWS_PER_TILE)])


@functools.partial(
    pl.kernel,
    out_type=(jax.ShapeDtypeStruct((NC, NP_, D_HID), jnp.float32),
              jax.ShapeDtypeStruct((NP_, D_HID), jnp.float32)),
    mesh=_mesh(),
    compiler_params=pltpu.CompilerParams(use_tc_tiling_on_sc=False,
                                         needs_layout_passes=False),
    scratch_types=[
        pltpu.VMEM((NCHUNK, 2, CB), jnp.int32),
        [pltpu.VMEM((CB, D_HID), jnp.float32)] * 8,
        pltpu.VMEM((CB, D_HID), jnp.float32),
        [pltpu.VMEM((ROWS_PER_TILE, D_HID), jnp.float32)] * 4,
        pltpu.VMEM((ROWS_PER_TILE, D_HID), jnp.float32),
        pltpu.VMEM((D_HID,), jnp.float32),
        pltpu.VMEM_SHARED((NP_, D_HID), jnp.float32),
        pltpu.VMEM_SHARED((NP_, D_HID), jnp.float32),
        [pltpu.SemaphoreType.DMA] * 8,
        [pltpu.SemaphoreType.DMA] * 8,
    ],
)
def _relu_agg_kernel(p_hbm, xws_hbm, dis16_hbm, b1_hbm, edges_hbm,
                     out_hbm, hs_hbm, idx_v, rows, zero_v, bufs,
                     hrow_v, b1_v, acc_sh, htab_sh, gsem, ssem):
    NSLOT, LA = 8, 4
    c = lax.axis_index("c")
    s = lax.axis_index("s")
    wid = s * NC + c
    row0 = s * ROWS_PER_TILE
    bp0, bp1, bxw, bdis = bufs

    for i in range(CB):
        zero_v[i, :] = jnp.zeros((D_HID,), jnp.float32)
    for i in range(ROWS_PER_TILE // CB):
        pltpu.sync_copy(zero_v, acc_sh.at[pl.ds(row0 + i * CB, CB)])

    pltpu.sync_copy(b1_hbm, b1_v)
    pltpu.sync_copy(p_hbm.at[0, pl.ds(row0, ROWS_PER_TILE)], bp0)
    pltpu.sync_copy(p_hbm.at[1, pl.ds(row0, ROWS_PER_TILE)], bp1)
    pltpu.sync_copy(xws_hbm.at[pl.ds(row0, ROWS_PER_TILE)], bxw)
    pltpu.sync_copy(dis16_hbm.at[pl.ds(row0, ROWS_PER_TILE)], bdis)
    b1r = b1_v[...]

    def rbody(r4, carry):
        for u in range(4):
            r = r4 * 4 + u
            d = bdis[r, :]
            t = (bp0[r, :] + bp1[r, :] + bxw[r, :]) * d + b1r
            hrow_v[r, :] = jnp.maximum(t, 0.0) * d
        return carry

    lax.fori_loop(0, ROWS_PER_TILE // 4, rbody, 0)
    pltpu.sync_copy(hrow_v, htab_sh.at[pl.ds(row0, ROWS_PER_TILE)])

    @pl.when(c == 0)
    def _():
        pltpu.sync_copy(hrow_v, hs_hbm.at[pl.ds(row0, ROWS_PER_TILE)])

    plsc.subcore_barrier()

    pltpu.sync_copy(edges_hbm.at[wid], idx_v)

    for j in range(LA):
        pltpu.async_copy(htab_sh.at[idx_v.at[j, 0]], rows[j], gsem[j])

    def body(g, carry):
        for b in range(NSLOT):
            j = g * NSLOT + b
            b2 = (b + LA) % NSLOT
            pltpu.make_async_copy(
                p_hbm.at[0, pl.ds(0, CB)], rows[b], gsem[b]).wait()
            pltpu.async_copy(rows[b], acc_sh.at[idx_v.at[j, 1]], ssem[b],
                             add=True)

            @pl.when((j >= NSLOT - LA) & (j < NCHUNK - LA))
            def _():
                pltpu.make_async_copy(
                    p_hbm.at[0, pl.ds(0, CB)], rows[b2], ssem[b2]).wait()

            @pl.when(j < NCHUNK - LA)
            def _():
                pltpu.async_copy(
                    htab_sh.at[idx_v.at[j + LA, 0]], rows[b2], gsem[b2])
        return carry

    lax.fori_loop(0, NCHUNK // NSLOT, body, 0)
    for b in range(NSLOT):
        pltpu.make_async_copy(
            p_hbm.at[0, pl.ds(0, CB)], rows[b], ssem[b]).wait()
    plsc.subcore_barrier()
    pltpu.sync_copy(acc_sh.at[pl.ds(row0, ROWS_PER_TILE)],
                    out_hbm.at[c, pl.ds(row0, ROWS_PER_TILE)])


def _mm1_body(x_ref, w1_ref, xw_ref):
    xw = jnp.dot(x_ref[...], w1_ref[...], preferred_element_type=jnp.float32)
    xw_ref[pl.ds(0, N), :] = xw
    xw_ref[pl.ds(N, NP_ - N), :] = jnp.zeros((NP_ - N, D_HID), jnp.float32)


def _k5_body(q_ref, hs_ref, dis16_ref, w2_ref, b2_ref, out_ref):
    z = (q_ref[0] + q_ref[1] + hs_ref[...]) * dis16_ref[...]
    out_ref[...] = (
        jnp.dot(z[:N], w2_ref[...], preferred_element_type=jnp.float32)
        + b2_ref[...][None, :]
    )


def kernel(x, edge_index, W1, b1, W2, b2):
    padblk = ((jnp.arange(2 * (E_PAD - E), dtype=jnp.int32)
               % (NP_ - N)) + N).reshape(2, E_PAD - E)
    epad = jnp.concatenate([edge_index, padblk], axis=1)
    edges = jnp.transpose(epad.reshape(2, E_PAD // CB, CB),
                          (1, 0, 2)).reshape(NW, NCHUNK, 2, CB)

    xw = pl.pallas_call(
        _mm1_body,
        out_shape=jax.ShapeDtypeStruct((NP_, D_HID), jnp.float32),
        in_specs=[pl.BlockSpec(memory_space=pltpu.VMEM)] * 2,
        out_specs=pl.BlockSpec(memory_space=pltpu.VMEM),
    )(x, W1)

    p, xw_s, dis16 = _pre_agg_kernel(xw, edges)

    q, h_s = _relu_agg_kernel(p, xw_s, dis16, b1, edges)

    out = pl.pallas_call(
        _k5_body,
        out_shape=jax.ShapeDtypeStruct((N, D_OUT), jnp.float32),
        in_specs=[pl.BlockSpec(memory_space=pltpu.VMEM)] * 5,
        out_specs=pl.BlockSpec(memory_space=pltpu.VMEM),
    )(q, h_s, dis16, W2, b2)

    return out
